# baseline (speedup 1.0000x reference)
import functools

import jax
import jax.numpy as jnp
from jax import lax
from jax.experimental import pallas as pl
from jax.experimental.pallas import tpu as pltpu
from jax.experimental.pallas import tpu_sc as plsc

_B = 4096
_R = 128
_L = 1000
_LP = 1024


def _tc_prep(params, resid):
    def body(resid_ref, p_ref, tld_ref):
        r = resid_ref[0, 0]
        x = p_ref[:, 0:1]
        y = p_ref[:, 1:2]
        z = -jnp.abs(p_ref[:, 2:3])
        inv = 1.0 / jnp.maximum(jnp.sqrt(x * x + y * y + z * z), 1e-12)
        bc = lambda v: jnp.broadcast_to(v[:, :, None], (_LP, 1, _R))
        xb, yb, zb = bc(x * inv), bc(y * inv), bc(z * inv)
        tld_ref[...] = jnp.concatenate([xb, yb, zb, zb], axis=1) + r

    return pl.pallas_call(
        body,
        in_specs=[
            pl.BlockSpec(memory_space=pltpu.SMEM),
            pl.BlockSpec((_LP, 4), lambda: (0, 0)),
        ],
        out_specs=pl.BlockSpec((_LP, 4, _R), lambda: (0, 0, 0)),
        out_shape=jax.ShapeDtypeStruct((_LP, 4, _R), jnp.float32),
    )(resid, params)


def _sc_gather_expanded(t_ld, idx):
    info = plsc.get_sparse_core_info()
    nc, ns = info.num_cores, info.num_subcores
    nw = nc * ns
    b_per_w = _B // nw
    rows_per_s = _LP // ns

    mesh = plsc.VectorSubcoreMesh(core_axis_name="c", subcore_axis_name="s")

    @functools.partial(
        pl.kernel,
        mesh=mesh,
        out_type=jax.ShapeDtypeStruct((_B, 4, _R), jnp.float32),
        scratch_types=[
            pltpu.VMEM((b_per_w // 2,), jnp.int32),
            pltpu.VMEM((b_per_w // 2,), jnp.int32),
            pltpu.VMEM((b_per_w // 2, 4, _R), jnp.float32),
            pltpu.VMEM((b_per_w // 2, 4, _R), jnp.float32),
            pltpu.VMEM_SHARED((_LP, 4, _R), jnp.float32),
            pltpu.SemaphoreType.DMA,
            pltpu.SemaphoreType.DMA,
        ],
    )
    def gather_kernel(tld_hbm, idx_hbm, old_hbm, idx0_v, idx1_v,
                      rows0_v, rows1_v, tab_s, sem, wsem):
        cid = lax.axis_index("c")
        sid = lax.axis_index("s")
        half = b_per_w // 2
        base = (sid * nc + cid) * b_per_w
        srow = sid * rows_per_s
        pltpu.sync_copy(tld_hbm.at[pl.ds(srow, rows_per_s)],
                        tab_s.at[pl.ds(srow, rows_per_s)])
        pltpu.sync_copy(idx_hbm.at[pl.ds(base, half)], idx0_v)
        pltpu.sync_copy(idx_hbm.at[pl.ds(base + half, half)], idx1_v)
        plsc.subcore_barrier()
        pltpu.async_copy(tab_s.at[idx0_v], rows0_v, sem).wait()
        w0 = pltpu.async_copy(rows0_v, old_hbm.at[pl.ds(base, half)], wsem)
        pltpu.async_copy(tab_s.at[idx1_v], rows1_v, sem).wait()
        w1 = pltpu.async_copy(rows1_v, old_hbm.at[pl.ds(base + half, half)],
                              wsem)
        w0.wait()
        w1.wait()

    return gather_kernel(t_ld, idx)


def _tc_intensity(idx_col, li_pad, resid):
    blk = 1024
    grid = (_B // blk,)

    def body(resid_ref, idx_ref, li_ref, out_ref):
        r = resid_ref[0, 0]
        iv = jnp.broadcast_to(idx_ref[...], (blk, _LP))
        onehot = jnp.where(
            iv == lax.broadcasted_iota(jnp.int32, (blk, _LP), 1),
            1.0, 0.0).astype(jnp.bfloat16)
        ia = jnp.abs(li_ref[...])
        hi = ia.astype(jnp.bfloat16)
        lo = (ia - hi.astype(jnp.float32)).astype(jnp.bfloat16)
        dims = (((1,), (0,)), ((), ()))
        g = (lax.dot_general(onehot, jnp.broadcast_to(hi, (_LP, _R)), dims,
                             preferred_element_type=jnp.float32)
             + lax.dot_general(onehot, jnp.broadcast_to(lo, (_LP, _R)), dims,
                               preferred_element_type=jnp.float32))
        out_ref[...] = jnp.broadcast_to(g[:, None, :], (blk, 4, _R)) + r

    return pl.pallas_call(
        body,
        grid=grid,
        in_specs=[
            pl.BlockSpec(memory_space=pltpu.SMEM),
            pl.BlockSpec((blk, 1), lambda i: (i, 0)),
            pl.BlockSpec((_LP, 1), lambda i: (0, 0)),
        ],
        out_specs=pl.BlockSpec((blk, 4, _R), lambda i: (i, 0, 0)),
        out_shape=jax.ShapeDtypeStruct((_B, 4, _R), jnp.float32),
    )(resid, idx_col, li_pad)


def _to_logical(o):
    o = o.transpose(0, 2, 1)
    return o.reshape(_B * _R, 4)[:, :3]


def kernel(light_direction_xy, light_direction_z, light_intensity, idx, num_rays):
    idx32 = idx.astype(jnp.int32)
    resid = (jnp.asarray(num_rays, jnp.float32) - _R).reshape(1, 1)

    params = jnp.pad(
        jnp.concatenate([light_direction_xy, light_direction_z], axis=1),
        ((0, _LP - _L), (0, 1)))
    t_ld = _tc_prep(params, resid)
    o_ld = _sc_gather_expanded(t_ld, idx32)

    idx_col = idx32.reshape(_B, 1)
    li_pad = jnp.pad(light_intensity, ((0, _LP - _L), (0, 0)))
    o_li = _tc_intensity(idx_col, li_pad, resid)

    return (_to_logical(o_ld), _to_logical(o_li))


# speedup vs baseline: 1.0055x; 1.0055x over previous
import functools

import jax
import jax.numpy as jnp
from jax import lax
from jax.experimental import pallas as pl
from jax.experimental.pallas import tpu as pltpu
from jax.experimental.pallas import tpu_sc as plsc

_B = 4096
_R = 128
_L = 1000
_LP = 1024


def _tc_prep(params, resid):
    def body(resid_ref, p_ref, tld_ref):
        r = resid_ref[0, 0]
        x = p_ref[:, 0:1]
        y = p_ref[:, 1:2]
        z = -jnp.abs(p_ref[:, 2:3])
        inv = 1.0 / jnp.maximum(jnp.sqrt(x * x + y * y + z * z), 1e-12)
        bc = lambda v: jnp.broadcast_to(v[:, :, None], (_LP, 1, _R))
        xb, yb, zb = bc(x * inv), bc(y * inv), bc(z * inv)
        tld_ref[...] = jnp.concatenate([xb, yb, zb, zb], axis=1) + r

    return pl.pallas_call(
        body,
        in_specs=[
            pl.BlockSpec(memory_space=pltpu.SMEM),
            pl.BlockSpec((_LP, 4), lambda: (0, 0)),
        ],
        out_specs=pl.BlockSpec((_LP, 4, _R), lambda: (0, 0, 0)),
        out_shape=jax.ShapeDtypeStruct((_LP, 4, _R), jnp.float32),
    )(resid, params)


def _sc_gather_expanded(t_ld, idx):
    info = plsc.get_sparse_core_info()
    nc, ns = info.num_cores, info.num_subcores
    nw = nc * ns
    b_per_w = _B // nw
    rows_per_s = _LP // ns

    mesh = plsc.VectorSubcoreMesh(core_axis_name="c", subcore_axis_name="s")

    @functools.partial(
        pl.kernel,
        mesh=mesh,
        out_type=jax.ShapeDtypeStruct((_B, 4, _R), jnp.float32),
        scratch_types=[
            pltpu.VMEM((b_per_w,), jnp.int32),
            pltpu.VMEM((b_per_w, 4, _R), jnp.float32),
            pltpu.VMEM_SHARED((_LP, 4, _R), jnp.float32),
            pltpu.SemaphoreType.DMA,
        ],
    )
    def gather_kernel(tld_hbm, idx_hbm, old_hbm, idx_v, rows_v, tab_s, sem):
        cid = lax.axis_index("c")
        sid = lax.axis_index("s")
        base = (sid * nc + cid) * b_per_w
        srow = sid * rows_per_s
        pltpu.sync_copy(tld_hbm.at[pl.ds(srow, rows_per_s)],
                        tab_s.at[pl.ds(srow, rows_per_s)])
        pltpu.sync_copy(idx_hbm.at[pl.ds(base, b_per_w)], idx_v)
        plsc.subcore_barrier()
        pltpu.async_copy(tab_s.at[idx_v], rows_v, sem).wait()
        pltpu.sync_copy(rows_v, old_hbm.at[pl.ds(base, b_per_w)])

    return gather_kernel(t_ld, idx)


def _tc_intensity(idx_col, li_pad, resid):
    blk = 1024
    grid = (_B // blk,)

    def body(resid_ref, idx_ref, li_ref, out_ref):
        r = resid_ref[0, 0]
        iv = jnp.broadcast_to(idx_ref[...], (blk, _LP))
        onehot = jnp.where(
            iv == lax.broadcasted_iota(jnp.int32, (blk, _LP), 1),
            1.0, 0.0).astype(jnp.bfloat16)
        ia = jnp.abs(li_ref[...])
        hi = ia.astype(jnp.bfloat16)
        lo = (ia - hi.astype(jnp.float32)).astype(jnp.bfloat16)
        dims = (((1,), (0,)), ((), ()))
        g = (lax.dot_general(onehot, jnp.broadcast_to(hi, (_LP, _R)), dims,
                             preferred_element_type=jnp.float32)
             + lax.dot_general(onehot, jnp.broadcast_to(lo, (_LP, _R)), dims,
                               preferred_element_type=jnp.float32))
        out_ref[...] = jnp.broadcast_to(g[:, None, :], (blk, 4, _R)) + r

    return pl.pallas_call(
        body,
        grid=grid,
        in_specs=[
            pl.BlockSpec(memory_space=pltpu.SMEM),
            pl.BlockSpec((blk, 1), lambda i: (i, 0)),
            pl.BlockSpec((_LP, 1), lambda i: (0, 0)),
        ],
        out_specs=pl.BlockSpec((blk, 4, _R), lambda i: (i, 0, 0)),
        out_shape=jax.ShapeDtypeStruct((_B, 4, _R), jnp.float32),
    )(resid, idx_col, li_pad)


def _to_logical(o):
    o = o.transpose(0, 2, 1)
    return o.reshape(_B * _R, 4)[:, :3]


def kernel(light_direction_xy, light_direction_z, light_intensity, idx, num_rays):
    idx32 = idx.astype(jnp.int32)
    resid = (jnp.asarray(num_rays, jnp.float32) - _R).reshape(1, 1)

    params = jnp.pad(
        jnp.concatenate([light_direction_xy, light_direction_z], axis=1),
        ((0, _LP - _L), (0, 1)))
    t_ld = _tc_prep(params, resid)
    o_ld = _sc_gather_expanded(t_ld, idx32)

    idx_col = idx32.reshape(_B, 1)
    li_pad = jnp.pad(light_intensity, ((0, _LP - _L), (0, 0)))
    o_li = _tc_intensity(idx_col, li_pad, resid)

    return (_to_logical(o_ld), _to_logical(o_li))
